# Initial kernel scaffold; baseline (speedup 1.0000x reference)
import jax
import jax.numpy as jnp
from jax.experimental import pallas as pl


def kernel(x, emb_content_id, emb_bundle_id, emb_cont_user_answer, emb_part, emb_tag, emb_lag_time, emb_elapsed_time, tag_wts, cont_wts, bn0_g, bn0_b, W_ih, W_hh, b_ih, b_hh, bn1_g, bn1_b, lin1_W, lin1_b, bn2_g, bn2_b, out_W, out_b):
    raise NotImplementedError("write your pallas kernel here")


# speedup vs baseline: 1.3639x; 1.3639x over previous
import functools

import jax
import jax.numpy as jnp
from jax import lax
from jax.experimental import pallas as pl
from jax.experimental.pallas import tpu as pltpu
from jax.experimental.pallas import tpu_sc as plsc

B = 1024
S = 200
NTOK = B * S
HID = 512
EW = 240

COL_CONTENT = 0
COL_BUNDLE = 32
COL_CUA = 64
COL_PART = 80
COL_TAG = 96
COL_LAG = 192
COL_ELA = 208
COL_CONT = 224

NC, NS = 2, 16
NW = NC * NS
TOK_PER_W = NTOK // NW
CHUNK = 1600
NCHUNK = TOK_PER_W // CHUNK


def _sc_gather_body(tab_c, tab_b, tab_q, tab_p, tab_t, tab_l, tab_e,
                    idx_c, idx_b, idx_q, idx_p, idx_t, idx_l, idx_e,
                    cont_sb, out, idx_v, rows32_v, rows16_v, sem):
    wid = lax.axis_index("s") * NC + lax.axis_index("c")
    for k in range(NCHUNK):
        base = wid * TOK_PER_W + k * CHUNK

        def gather(idx_slice, tab, col, buf, w):
            pltpu.sync_copy(idx_slice, idx_v)
            pltpu.async_copy(tab.at[idx_v], buf, sem).wait()
            pltpu.sync_copy(buf, out.at[pl.ds(base, CHUNK), pl.ds(col, w)])

        gather(idx_c.at[pl.ds(base, CHUNK)], tab_c, COL_CONTENT, rows32_v, 32)
        gather(idx_b.at[pl.ds(base, CHUNK)], tab_b, COL_BUNDLE, rows32_v, 32)
        gather(idx_q.at[pl.ds(base, CHUNK)], tab_q, COL_CUA, rows16_v, 16)
        gather(idx_p.at[pl.ds(base, CHUNK)], tab_p, COL_PART, rows16_v, 16)
        for j in range(6):
            gather(idx_t.at[j, pl.ds(base, CHUNK)], tab_t, COL_TAG + 16 * j,
                   rows16_v, 16)
        gather(idx_l.at[pl.ds(base, CHUNK)], tab_l, COL_LAG, rows16_v, 16)
        gather(idx_e.at[pl.ds(base, CHUNK)], tab_e, COL_ELA, rows16_v, 16)
        pltpu.sync_copy(cont_sb.at[pl.ds(base, CHUNK), :], rows16_v)
        pltpu.sync_copy(rows16_v, out.at[pl.ds(base, CHUNK), pl.ds(COL_CONT, 16)])


def _build_E(tabs, idxs, cont_sb):
    return pl.kernel(
        _sc_gather_body,
        out_type=jax.ShapeDtypeStruct((NTOK, EW), jnp.float32),
        mesh=plsc.VectorSubcoreMesh(core_axis_name="c", subcore_axis_name="s"),
        scratch_types=[
            pltpu.VMEM((CHUNK,), jnp.int32),
            pltpu.VMEM((CHUNK, 32), jnp.float32),
            pltpu.VMEM((CHUNK, 16), jnp.float32),
            pltpu.SemaphoreType.DMA,
        ],
        compiler_params=pltpu.CompilerParams(use_tc_tiling_on_sc=False),
    )(*tabs, *idxs, cont_sb)


ST_CH = 8192
ST_N = NTOK // ST_CH


def _stats_body(cont_blk, ag, bg, scale_out, shift_out, acc):
    i = pl.program_id(0)

    @pl.when(i == 0)
    def _():
        acc[...] = jnp.zeros_like(acc)

    blk = cont_blk[...]
    acc[0:1, :] += jnp.sum(blk, axis=0, keepdims=True)
    acc[1:2, :] += jnp.sum(blk * blk, axis=0, keepdims=True)

    @pl.when(i == ST_N - 1)
    def _():
        n = jnp.float32(NTOK)
        m0 = acc[0:1, :] / n
        v0 = acc[1:2, :] / n - m0 * m0
        a = ag[...] * lax.rsqrt(v0 + 1e-5)
        bsh = bg[...] - m0 * a
        one = jnp.ones((1, COL_CONT), jnp.float32)
        zero = jnp.zeros((1, COL_CONT), jnp.float32)
        tail1 = jnp.ones((1, 8), jnp.float32)
        tail0 = jnp.zeros((1, 8), jnp.float32)
        scale_out[...] = jnp.concatenate([one, a, tail1], axis=1)
        shift_out[...] = jnp.concatenate([zero, bsh, tail0], axis=1)


def _bn0_rows(cont2d, ag, bg):
    return pl.pallas_call(
        _stats_body,
        grid=(ST_N,),
        in_specs=[
            pl.BlockSpec((ST_CH, 8), lambda i: (i, 0)),
            pl.BlockSpec((1, 8), lambda i: (0, 0)),
            pl.BlockSpec((1, 8), lambda i: (0, 0)),
        ],
        out_specs=[
            pl.BlockSpec((1, EW), lambda i: (0, 0)),
            pl.BlockSpec((1, EW), lambda i: (0, 0)),
        ],
        out_shape=[
            jax.ShapeDtypeStruct((1, EW), jnp.float32),
            jax.ShapeDtypeStruct((1, EW), jnp.float32),
        ],
        scratch_shapes=[pltpu.VMEM((2, 8), jnp.float32)],
        compiler_params=pltpu.CompilerParams(
            dimension_semantics=("arbitrary",)),
    )(cont2d, ag, bg)


def _lstm_body(E_blk, scale_r, shift_r, Wp, Whh, bias,
               bn1g, bn1b, l1w, l1b, bn2g, bn2b, ow, ob,
               out, h_ref, c_ref):
    t = pl.program_id(0)

    @pl.when(t == 0)
    def _():
        h_ref[...] = jnp.zeros_like(h_ref)
        c_ref[...] = jnp.zeros_like(c_ref)

    xt = E_blk[0] * scale_r[...] + shift_r[...]
    g = jnp.dot(xt, Wp[...], preferred_element_type=jnp.float32)
    g = g + jnp.dot(h_ref[...], Whh[...], preferred_element_type=jnp.float32)
    g = g + bias[...]
    i_ = jax.nn.sigmoid(g[:, 0:HID])
    f_ = jax.nn.sigmoid(g[:, HID:2 * HID])
    g_ = jnp.tanh(g[:, 2 * HID:3 * HID])
    o_ = jax.nn.sigmoid(g[:, 3 * HID:4 * HID])
    c = f_ * c_ref[...] + i_ * g_
    h = o_ * jnp.tanh(c)
    c_ref[...] = c
    h_ref[...] = h

    @pl.when(t == S - 1)
    def _():
        m1 = jnp.mean(h, axis=0, keepdims=True)
        d1 = h - m1
        v1 = jnp.mean(d1 * d1, axis=0, keepdims=True)
        hid = d1 * lax.rsqrt(v1 + 1e-5) * bn1g[...] + bn1b[...]
        hid = jnp.maximum(
            jnp.dot(hid, l1w[...], preferred_element_type=jnp.float32)
            + l1b[...], 0.0)
        m2 = jnp.mean(hid, axis=0, keepdims=True)
        d2 = hid - m2
        v2 = jnp.mean(d2 * d2, axis=0, keepdims=True)
        hid = d2 * lax.rsqrt(v2 + 1e-5) * bn2g[...] + bn2b[...]
        out[...] = jnp.sum(hid * ow[...], axis=1, keepdims=True) + ob[...]


def _run_lstm(E3, scale_r, shift_r, Wp, Whh, bias,
              bn1g, bn1b, l1w, l1b, bn2g, bn2b, ow, ob):
    const = lambda shp: pl.BlockSpec(shp, lambda t: tuple(0 for _ in shp))
    return pl.pallas_call(
        _lstm_body,
        grid=(S,),
        in_specs=[
            pl.BlockSpec((1, B, EW), lambda t: (t, 0, 0)),
            const((1, EW)), const((1, EW)),
            const((EW, 4 * HID)), const((HID, 4 * HID)), const((1, 4 * HID)),
            const((1, HID)), const((1, HID)),
            const((HID, HID // 2)), const((1, HID // 2)),
            const((1, HID // 2)), const((1, HID // 2)),
            const((1, HID // 2)), const((1, 1)),
        ],
        out_specs=pl.BlockSpec((B, 1), lambda t: (0, 0)),
        out_shape=jax.ShapeDtypeStruct((B, 1), jnp.float32),
        scratch_shapes=[
            pltpu.VMEM((B, HID), jnp.float32),
            pltpu.VMEM((B, HID), jnp.float32),
        ],
        compiler_params=pltpu.CompilerParams(
            dimension_semantics=("arbitrary",)),
    )(E3, scale_r, shift_r, Wp, Whh, bias,
      bn1g, bn1b, l1w, l1b, bn2g, bn2b, ow, ob)


def kernel(x, emb_content_id, emb_bundle_id, emb_cont_user_answer, emb_part,
           emb_tag, emb_lag_time, emb_elapsed_time, tag_wts, cont_wts,
           bn0_g, bn0_b, W_ih, W_hh, b_ih, b_hh, bn1_g, bn1_b,
           lin1_W, lin1_b, bn2_g, bn2_b, out_W, out_b):
    f32 = jnp.float32
    xi = x.astype(jnp.int32)

    def tm(col):
        return xi[:, :, col].T.reshape(-1)

    idx_c = tm(0)
    idx_b = tm(1)
    idx_q = tm(2)
    idx_p = tm(3)
    idx_t = jnp.stack([tm(4 + j) for j in range(6)], axis=0)
    idx_l = tm(10)
    idx_e = tm(11)

    pad16 = lambda t: jnp.pad(t, ((0, 0), (0, 16 - t.shape[1])))
    tabs = (emb_content_id, emb_bundle_id, pad16(emb_cont_user_answer),
            pad16(emb_part), emb_tag, emb_lag_time, emb_elapsed_time)

    cont_sb = jnp.pad(
        jnp.swapaxes(x[:, :, 12:20], 0, 1).reshape(NTOK, 8),
        ((0, 0), (0, 8)))

    E = _build_E(tabs, (idx_c, idx_b, idx_q, idx_p, idx_t, idx_l, idx_e),
                 cont_sb)
    E3 = E.reshape(S, B, EW)

    cont2d = x[:, :, 12:20].reshape(NTOK, 8)
    ag = (bn0_g * cont_wts).reshape(1, 8)
    bg = (bn0_b * cont_wts).reshape(1, 8)
    scale_r, shift_r = _bn0_rows(cont2d, ag, bg)

    WT = W_ih.T
    z = lambda n: jnp.zeros((n, 4 * HID), f32)
    Wp = jnp.concatenate(
        [WT[0:64],
         WT[64:69], z(11),
         WT[69:73], z(12),
         ] + [WT[73:89] * tag_wts[j][:, None] for j in range(6)]
        + [WT[89:121],
           WT[121:129], z(8)],
        axis=0)
    Whh = W_hh.T
    bias = (b_ih + b_hh).reshape(1, 4 * HID)

    out = _run_lstm(
        E3, scale_r, shift_r, Wp, Whh, bias,
        bn1_g.reshape(1, HID), bn1_b.reshape(1, HID),
        lin1_W.T, lin1_b.reshape(1, HID // 2),
        bn2_g.reshape(1, HID // 2), bn2_b.reshape(1, HID // 2),
        out_W.reshape(1, HID // 2), out_b.reshape(1, 1))
    return out.reshape(-1)


# speedup vs baseline: 1.4420x; 1.0573x over previous
import functools

import jax
import jax.numpy as jnp
from jax import lax
from jax.experimental import pallas as pl
from jax.experimental.pallas import tpu as pltpu
from jax.experimental.pallas import tpu_sc as plsc

B = 1024
S = 200
NTOK = B * S
HID = 512
NROW = 15
EW = NROW * 16
COL_CONT = 224

NC, NS = 2, 16
NW = NC * NS
TOK_PER_W = NTOK // NW
CHUNK = 200
NCHUNK = TOK_PER_W // CHUNK
ROWS = CHUNK * NROW


def _sc_gather_body(supertab, idx_all, out, idx_v0, idx_v1, row_v0, row_v1,
                    gsem, wsem):
    wid = lax.axis_index("s") * NC + lax.axis_index("c")
    idxb = (idx_v0, idx_v1)
    rowb = (row_v0, row_v1)
    gathers = [None, None]
    writes = []
    bases = []
    for k in range(NCHUNK):
        b = k % 2
        rbase = (wid * TOK_PER_W + k * CHUNK) * NROW
        bases.append(rbase)
        if k >= 2:
            writes[k - 2].wait()
        pltpu.sync_copy(idx_all.at[pl.ds(rbase, ROWS)], idxb[b])
        gathers[b] = pltpu.async_copy(supertab.at[idxb[b]], rowb[b], gsem)
        if k >= 1:
            gathers[1 - b].wait()
            writes.append(pltpu.async_copy(
                rowb[1 - b], out.at[pl.ds(bases[k - 1], ROWS)], wsem))
    last = (NCHUNK - 1) % 2
    gathers[last].wait()
    writes.append(pltpu.async_copy(
        rowb[last], out.at[pl.ds(bases[NCHUNK - 1], ROWS)], wsem))
    writes[-2].wait()
    writes[-1].wait()


def _build_E(supertab, idx_all):
    return pl.kernel(
        _sc_gather_body,
        out_type=jax.ShapeDtypeStruct((NTOK * NROW, 16), jnp.float32),
        mesh=plsc.VectorSubcoreMesh(core_axis_name="c", subcore_axis_name="s"),
        scratch_types=[
            pltpu.VMEM((ROWS,), jnp.int32),
            pltpu.VMEM((ROWS,), jnp.int32),
            pltpu.VMEM((ROWS, 16), jnp.float32),
            pltpu.VMEM((ROWS, 16), jnp.float32),
            pltpu.SemaphoreType.DMA,
            pltpu.SemaphoreType.DMA,
        ],
        compiler_params=pltpu.CompilerParams(use_tc_tiling_on_sc=False),
    )(supertab, idx_all)


ST_CH = 8192
ST_N = NTOK // ST_CH


def _stats_body(cont_blk, ag, bg, scale_out, shift_out, acc):
    i = pl.program_id(0)

    @pl.when(i == 0)
    def _():
        acc[...] = jnp.zeros_like(acc)

    blk = cont_blk[...]
    acc[0:1, :] += jnp.sum(blk, axis=0, keepdims=True)
    acc[1:2, :] += jnp.sum(blk * blk, axis=0, keepdims=True)

    @pl.when(i == ST_N - 1)
    def _():
        n = jnp.float32(NTOK)
        m0 = acc[0:1, :] / n
        v0 = acc[1:2, :] / n - m0 * m0
        a = ag[...] * lax.rsqrt(v0 + 1e-5)
        bsh = bg[...] - m0 * a
        one = jnp.ones((1, COL_CONT), jnp.float32)
        zero = jnp.zeros((1, COL_CONT), jnp.float32)
        tail1 = jnp.ones((1, 8), jnp.float32)
        tail0 = jnp.zeros((1, 8), jnp.float32)
        scale_out[...] = jnp.concatenate([one, a, tail1], axis=1)
        shift_out[...] = jnp.concatenate([zero, bsh, tail0], axis=1)


def _bn0_rows(cont2d, ag, bg):
    return pl.pallas_call(
        _stats_body,
        grid=(ST_N,),
        in_specs=[
            pl.BlockSpec((ST_CH, 8), lambda i: (i, 0)),
            pl.BlockSpec((1, 8), lambda i: (0, 0)),
            pl.BlockSpec((1, 8), lambda i: (0, 0)),
        ],
        out_specs=[
            pl.BlockSpec((1, EW), lambda i: (0, 0)),
            pl.BlockSpec((1, EW), lambda i: (0, 0)),
        ],
        out_shape=[
            jax.ShapeDtypeStruct((1, EW), jnp.float32),
            jax.ShapeDtypeStruct((1, EW), jnp.float32),
        ],
        scratch_shapes=[pltpu.VMEM((2, 8), jnp.float32)],
        compiler_params=pltpu.CompilerParams(
            dimension_semantics=("arbitrary",)),
    )(cont2d, ag, bg)


def _lstm_body(E_blk, scale_r, shift_r, Wp, Whh, bias,
               bn1g, bn1b, l1w, l1b, bn2g, bn2b, ow, ob,
               out, h_ref, c_ref):
    t = pl.program_id(0)

    @pl.when(t == 0)
    def _():
        h_ref[...] = jnp.zeros_like(h_ref)
        c_ref[...] = jnp.zeros_like(c_ref)

    xt = E_blk[0] * scale_r[...] + shift_r[...]
    g = jnp.dot(xt, Wp[...], preferred_element_type=jnp.float32)
    g = g + jnp.dot(h_ref[...], Whh[...], preferred_element_type=jnp.float32)
    g = g + bias[...]
    i_ = jax.nn.sigmoid(g[:, 0:HID])
    f_ = jax.nn.sigmoid(g[:, HID:2 * HID])
    g_ = jnp.tanh(g[:, 2 * HID:3 * HID])
    o_ = jax.nn.sigmoid(g[:, 3 * HID:4 * HID])
    c = f_ * c_ref[...] + i_ * g_
    h = o_ * jnp.tanh(c)
    c_ref[...] = c
    h_ref[...] = h

    @pl.when(t == S - 1)
    def _():
        m1 = jnp.mean(h, axis=0, keepdims=True)
        d1 = h - m1
        v1 = jnp.mean(d1 * d1, axis=0, keepdims=True)
        hid = d1 * lax.rsqrt(v1 + 1e-5) * bn1g[...] + bn1b[...]
        hid = jnp.maximum(
            jnp.dot(hid, l1w[...], preferred_element_type=jnp.float32)
            + l1b[...], 0.0)
        m2 = jnp.mean(hid, axis=0, keepdims=True)
        d2 = hid - m2
        v2 = jnp.mean(d2 * d2, axis=0, keepdims=True)
        hid = d2 * lax.rsqrt(v2 + 1e-5) * bn2g[...] + bn2b[...]
        out[...] = jnp.sum(hid * ow[...], axis=1, keepdims=True) + ob[...]


def _run_lstm(E3, scale_r, shift_r, Wp, Whh, bias,
              bn1g, bn1b, l1w, l1b, bn2g, bn2b, ow, ob):
    const = lambda shp: pl.BlockSpec(shp, lambda t: tuple(0 for _ in shp))
    return pl.pallas_call(
        _lstm_body,
        grid=(S,),
        in_specs=[
            pl.BlockSpec((1, B, EW), lambda t: (t, 0, 0)),
            const((1, EW)), const((1, EW)),
            const((EW, 4 * HID)), const((HID, 4 * HID)), const((1, 4 * HID)),
            const((1, HID)), const((1, HID)),
            const((HID, HID // 2)), const((1, HID // 2)),
            const((1, HID // 2)), const((1, HID // 2)),
            const((1, HID // 2)), const((1, 1)),
        ],
        out_specs=pl.BlockSpec((B, 1), lambda t: (0, 0)),
        out_shape=jax.ShapeDtypeStruct((B, 1), jnp.float32),
        scratch_shapes=[
            pltpu.VMEM((B, HID), jnp.float32),
            pltpu.VMEM((B, HID), jnp.float32),
        ],
        compiler_params=pltpu.CompilerParams(
            dimension_semantics=("arbitrary",)),
    )(E3, scale_r, shift_r, Wp, Whh, bias,
      bn1g, bn1b, l1w, l1b, bn2g, bn2b, ow, ob)


def kernel(x, emb_content_id, emb_bundle_id, emb_cont_user_answer, emb_part,
           emb_tag, emb_lag_time, emb_elapsed_time, tag_wts, cont_wts,
           bn0_g, bn0_b, W_ih, W_hh, b_ih, b_hh, bn1_g, bn1_b,
           lin1_W, lin1_b, bn2_g, bn2_b, out_W, out_b):
    f32 = jnp.float32
    xi = x.astype(jnp.int32)

    cont_sb = jnp.pad(
        jnp.swapaxes(x[:, :, 12:20], 0, 1).reshape(NTOK, 8),
        ((0, 0), (0, 8)))

    pad16 = lambda t: jnp.pad(t, ((0, 0), (0, 16 - t.shape[1])))
    sup_parts = [emb_content_id.reshape(-1, 16),
                 emb_bundle_id.reshape(-1, 16),
                 pad16(emb_cont_user_answer),
                 pad16(emb_part),
                 emb_tag,
                 emb_lag_time,
                 emb_elapsed_time,
                 cont_sb]
    offs = [0]
    for p in sup_parts:
        offs.append(offs[-1] + p.shape[0])
    supertab = jnp.concatenate(sup_parts, axis=0)

    def tm(col):
        return xi[:, :, col].T.reshape(-1)

    c = tm(0)
    bu = tm(1)
    cols = [2 * c, 2 * c + 1,
            offs[1] + 2 * bu, offs[1] + 2 * bu + 1,
            offs[2] + tm(2), offs[3] + tm(3)]
    cols += [offs[4] + tm(4 + j) for j in range(6)]
    cols += [offs[5] + tm(10), offs[6] + tm(11),
             offs[7] + jnp.arange(NTOK, dtype=jnp.int32)]
    idx_all = jnp.stack(cols, axis=1).reshape(-1)

    E = _build_E(supertab, idx_all)
    E3 = E.reshape(S, B, EW)

    cont2d = x[:, :, 12:20].reshape(NTOK, 8)
    ag = (bn0_g * cont_wts).reshape(1, 8)
    bg = (bn0_b * cont_wts).reshape(1, 8)
    scale_r, shift_r = _bn0_rows(cont2d, ag, bg)

    WT = W_ih.T
    z = lambda n: jnp.zeros((n, 4 * HID), f32)
    Wp = jnp.concatenate(
        [WT[0:64],
         WT[64:69], z(11),
         WT[69:73], z(12),
         ] + [WT[73:89] * tag_wts[j][:, None] for j in range(6)]
        + [WT[89:121],
           WT[121:129], z(8)],
        axis=0)
    Whh = W_hh.T
    bias = (b_ih + b_hh).reshape(1, 4 * HID)

    out = _run_lstm(
        E3, scale_r, shift_r, Wp, Whh, bias,
        bn1_g.reshape(1, HID), bn1_b.reshape(1, HID),
        lin1_W.T, lin1_b.reshape(1, HID // 2),
        bn2_g.reshape(1, HID // 2), bn2_b.reshape(1, HID // 2),
        out_W.reshape(1, HID // 2), out_b.reshape(1, 1))
    return out.reshape(-1)
